# Initial kernel scaffold; baseline (speedup 1.0000x reference)
import jax
import jax.numpy as jnp
from jax.experimental import pallas as pl


def kernel(x, edge_index, actions, W1, b1, W2, b2, W3, b3, g1, be1, g2, be2, g3, be3, fcW1, fcb1, fcW2, fcb2, fcW3, fcb3):
    raise NotImplementedError("write your pallas kernel here")


# speedup vs baseline: 13.6611x; 13.6611x over previous
import functools
import math

import jax
import jax.numpy as jnp
from jax import lax
from jax.experimental import pallas as pl
from jax.experimental.pallas import tpu as pltpu
from jax.experimental.pallas import tpu_sc as plsc

N = 10000
D = 128
E = 320000

NC = 2
NS = 16
NW = NC * NS

CHUNK = 128
CHUNKS_TOTAL = E // CHUNK
CHUNKS_PER_TILE = CHUNKS_TOTAL // NW
EXTRA = CHUNKS_TOTAL - CHUNKS_PER_TILE * NW
ROWS_A = 624
TAIL_LO = NS * ROWS_A
TAIL_N = N - TAIL_LO

BN_INV = 1.0 / math.sqrt(1.0 + 1e-5)

@functools.cache
def _mesh():
    return plsc.VectorSubcoreMesh(core_axis_name="c", subcore_axis_name="s")


def _sc_deg_body(dst_hbm, zeros_hbm, ones_hbm, out_hbm, idx_v, ones_v, sem,
                 shared):
    cid = lax.axis_index("c")
    sid = lax.axis_index("s")
    wid = sid * NC + cid
    pltpu.sync_copy(ones_hbm, ones_v)
    del sem

    if True:
        pltpu.sync_copy(zeros_hbm.at[pl.ds(sid * ROWS_A, ROWS_A)],
                        shared.at[pl.ds(sid * ROWS_A, ROWS_A)])

        @pl.when(sid == NS - 1)
        def _():
            pltpu.sync_copy(zeros_hbm.at[pl.ds(TAIL_LO, TAIL_N)],
                            shared.at[pl.ds(TAIL_LO, TAIL_N)])

        plsc.subcore_barrier()

        def body(k, carry):
            base = (wid * CHUNKS_PER_TILE + k) * CHUNK
            pltpu.sync_copy(dst_hbm.at[pl.ds(base, CHUNK)], idx_v)
            pltpu.sync_copy(ones_v, shared.at[idx_v], add=True)
            return carry

        lax.fori_loop(0, CHUNKS_PER_TILE, body, 0)

        @pl.when(wid < EXTRA)
        def _():
            base = (CHUNKS_PER_TILE * NW + wid) * CHUNK
            pltpu.sync_copy(dst_hbm.at[pl.ds(base, CHUNK)], idx_v)
            pltpu.sync_copy(ones_v, shared.at[idx_v], add=True)

        plsc.subcore_barrier()
        pltpu.sync_copy(shared.at[pl.ds(sid * ROWS_A, ROWS_A)],
                        out_hbm.at[cid, pl.ds(sid * ROWS_A, ROWS_A)])

        @pl.when(sid == NS - 1)
        def _():
            pltpu.sync_copy(shared.at[pl.ds(TAIL_LO, TAIL_N)],
                            out_hbm.at[cid, pl.ds(TAIL_LO, TAIL_N)])


@functools.cache
def _sc_deg():
    return pl.kernel(
        _sc_deg_body,
        out_type=jax.ShapeDtypeStruct((NC, N, 16), jnp.float32),
        mesh=_mesh(),
        scratch_types=[
            pltpu.VMEM((CHUNK,), jnp.int32),
            pltpu.VMEM((CHUNK, 16), jnp.float32),
            pltpu.SemaphoreType.DMA,
            pltpu.VMEM_SHARED((N, 16), jnp.float32),
        ],
    )


def _sc_agg_body(xs_hbm, src_hbm, dst_hbm, zeros_hbm, out_hbm,
                 isrc_v, idst_v, rows_v, sem, shared):
    cid = lax.axis_index("c")
    sid = lax.axis_index("s")
    wid = sid * NC + cid

    if True:
        pltpu.sync_copy(zeros_hbm.at[pl.ds(sid * ROWS_A, ROWS_A)],
                        shared.at[pl.ds(sid * ROWS_A, ROWS_A)])

        @pl.when(sid == NS - 1)
        def _():
            pltpu.sync_copy(zeros_hbm.at[pl.ds(TAIL_LO, TAIL_N)],
                            shared.at[pl.ds(TAIL_LO, TAIL_N)])

        plsc.subcore_barrier()

        def body(k, carry):
            base = (wid * CHUNKS_PER_TILE + k) * CHUNK
            pltpu.sync_copy(src_hbm.at[pl.ds(base, CHUNK)], isrc_v)
            pltpu.sync_copy(dst_hbm.at[pl.ds(base, CHUNK)], idst_v)
            pltpu.async_copy(xs_hbm.at[isrc_v], rows_v, sem).wait()
            pltpu.sync_copy(rows_v, shared.at[idst_v], add=True)
            return carry

        lax.fori_loop(0, CHUNKS_PER_TILE, body, 0)

        @pl.when(wid < EXTRA)
        def _():
            base = (CHUNKS_PER_TILE * NW + wid) * CHUNK
            pltpu.sync_copy(src_hbm.at[pl.ds(base, CHUNK)], isrc_v)
            pltpu.sync_copy(dst_hbm.at[pl.ds(base, CHUNK)], idst_v)
            pltpu.async_copy(xs_hbm.at[isrc_v], rows_v, sem).wait()
            pltpu.sync_copy(rows_v, shared.at[idst_v], add=True)

        plsc.subcore_barrier()
        pltpu.sync_copy(shared.at[pl.ds(sid * ROWS_A, ROWS_A)],
                        out_hbm.at[cid, pl.ds(sid * ROWS_A, ROWS_A)])

        @pl.when(sid == NS - 1)
        def _():
            pltpu.sync_copy(shared.at[pl.ds(TAIL_LO, TAIL_N)],
                            out_hbm.at[cid, pl.ds(TAIL_LO, TAIL_N)])


@functools.cache
def _sc_agg():
    return pl.kernel(
        _sc_agg_body,
        out_type=jax.ShapeDtypeStruct((NC, N, D), jnp.float32),
        mesh=_mesh(),
        scratch_types=[
            pltpu.VMEM((CHUNK,), jnp.int32),
            pltpu.VMEM((CHUNK,), jnp.int32),
            pltpu.VMEM((CHUNK, D), jnp.float32),
            pltpu.SemaphoreType.DMA,
            pltpu.VMEM_SHARED((N, D), jnp.float32),
        ],
    )


_RB = 1000
_GRID = N // _RB


def _dis(deg_ref):
    return lax.rsqrt(deg_ref[0, :, 0:1] + deg_ref[1, :, 0:1] + 1.0)


def _xw_body(x_ref, w_ref, deg_ref, o_ref):
    o_ref[...] = jnp.dot(x_ref[...], w_ref[...],
                         preferred_element_type=jnp.float32) * _dis(deg_ref)


def _fuse_body(p_ref, xs_ref, deg_ref, b_ref, g_ref, be_ref, w_ref, o_ref):
    dis = _dis(deg_ref)
    agg = (p_ref[0] + p_ref[1] + xs_ref[...]) * dis + b_ref[...]
    h = jnp.maximum(agg, 0.0) * (g_ref[...] * BN_INV) + be_ref[...]
    o_ref[...] = jnp.dot(h, w_ref[...],
                         preferred_element_type=jnp.float32) * dis


def _final_body(p_ref, xs_ref, deg_ref, b_ref, act_ref, fw1_ref, fb1_ref,
                g3_ref, be3_ref, fw2_ref, fb2_ref, fw3_ref, fb3_ref,
                o_ref, acc):
    i = pl.program_id(0)
    dis = _dis(deg_ref)
    h = jnp.maximum((p_ref[0] + p_ref[1] + xs_ref[...]) * dis + b_ref[...], 0.0)
    s = jnp.sum(h, axis=0, keepdims=True)

    @pl.when(i == 0)
    def _():
        acc[...] = s

    @pl.when(i > 0)
    def _():
        acc[...] = acc[...] + s

    @pl.when(i == _GRID - 1)
    def _():
        pooled = jnp.concatenate(
            [acc[...] * (1.0 / N), jnp.zeros((7, 128), jnp.float32)], axis=0)
        z = (jnp.dot(pooled, fw1_ref[0:128, :], preferred_element_type=jnp.float32)
             + jnp.dot(act_ref[...], fw1_ref[128:144, :],
                       preferred_element_type=jnp.float32)
             + fb1_ref[...])
        z = jnp.maximum(z, 0.0) * (g3_ref[...] * BN_INV) + be3_ref[...]
        z = jnp.maximum(jnp.dot(z, fw2_ref[...],
                                preferred_element_type=jnp.float32)
                        + fb2_ref[...], 0.0)
        z = jnp.dot(z, fw3_ref[...], preferred_element_type=jnp.float32) + fb3_ref[...]
        o_ref[...] = z[0:1, :]


def _full(shape):
    nd = len(shape)
    return pl.BlockSpec(shape, lambda i, _n=nd: (0,) * _n)


_row_spec = pl.BlockSpec((_RB, D), lambda i: (i, 0))
_p_spec = pl.BlockSpec((NC, _RB, D), lambda i: (0, i, 0))
_deg_spec = pl.BlockSpec((NC, _RB, 16), lambda i: (0, i, 0))

_tc_xw = pl.pallas_call(
    _xw_body,
    grid=(_GRID,),
    in_specs=[_row_spec, _full((D, D)), _deg_spec],
    out_specs=_row_spec,
    out_shape=jax.ShapeDtypeStruct((N, D), jnp.float32),
)

_tc_fuse = pl.pallas_call(
    _fuse_body,
    grid=(_GRID,),
    in_specs=[_p_spec, _row_spec, _deg_spec, _full((1, D)), _full((1, D)),
              _full((1, D)), _full((D, D))],
    out_specs=_row_spec,
    out_shape=jax.ShapeDtypeStruct((N, D), jnp.float32),
)

_tc_final = pl.pallas_call(
    _final_body,
    grid=(_GRID,),
    in_specs=[_p_spec, _row_spec, _deg_spec, _full((1, D)), _full((8, 16)),
              _full((303, D)), _full((1, D)), _full((1, D)), _full((1, D)),
              _full((D, 64)), _full((1, 64)), _full((64, 10)), _full((1, 10))],
    out_specs=_full((1, 10)),
    out_shape=jax.ShapeDtypeStruct((1, 10), jnp.float32),
    scratch_shapes=[pltpu.VMEM((1, D), jnp.float32)],
)


def kernel(x, edge_index, actions, W1, b1, W2, b2, W3, b3, g1, be1, g2, be2,
           g3, be3, fcW1, fcb1, fcW2, fcb2, fcW3, fcb3):
    src = edge_index[0]
    dst = edge_index[1]
    zeros_deg = jnp.zeros((N, 16), jnp.float32)
    ones_chunk = jnp.ones((CHUNK, 16), jnp.float32)
    zeros_big = jnp.zeros((N, D), jnp.float32)
    act8 = jnp.pad(actions.astype(jnp.float32), ((0, 7), (0, 0)))
    row = lambda v: v.reshape(1, -1)

    degp = _sc_deg()(dst, zeros_deg, ones_chunk)

    xs1 = _tc_xw(x, W1, degp)
    p1 = _sc_agg()(xs1, src, dst, zeros_big)
    xs2 = _tc_fuse(p1, xs1, degp, row(b1), row(g1), row(be1), W2)
    p2 = _sc_agg()(xs2, src, dst, zeros_big)
    xs3 = _tc_fuse(p2, xs2, degp, row(b2), row(g2), row(be2), W3)
    p3 = _sc_agg()(xs3, src, dst, zeros_big)

    return _tc_final(p3, xs3, degp, row(b3), act8, fcW1, row(fcb1),
                     row(g3), row(be3), fcW2, row(fcb2), fcW3, row(fcb3))


# speedup vs baseline: 15.5206x; 1.1361x over previous
import functools
import math

import jax
import jax.numpy as jnp
from jax import lax
from jax.experimental import pallas as pl
from jax.experimental.pallas import tpu as pltpu
from jax.experimental.pallas import tpu_sc as plsc

N = 10000
D = 128
E = 320000

NC = 2
NS = 16
NW = NC * NS

CHUNK = 128
CHUNKS_TOTAL = E // CHUNK
CHUNKS_PER_TILE = CHUNKS_TOTAL // NW
EXTRA = CHUNKS_TOTAL - CHUNKS_PER_TILE * NW
ROWS_A = 624
TAIL_LO = NS * ROWS_A
TAIL_N = N - TAIL_LO

BN_INV = 1.0 / math.sqrt(1.0 + 1e-5)

@functools.cache
def _mesh():
    return plsc.VectorSubcoreMesh(core_axis_name="c", subcore_axis_name="s")


def _sc_deg_body(dst_hbm, zeros_hbm, ones_hbm, out_hbm, idx_v, ones_v, sem,
                 shared):
    cid = lax.axis_index("c")
    sid = lax.axis_index("s")
    wid = sid * NC + cid
    pltpu.sync_copy(ones_hbm, ones_v)
    del sem

    if True:
        pltpu.sync_copy(zeros_hbm.at[pl.ds(sid * ROWS_A, ROWS_A)],
                        shared.at[pl.ds(sid * ROWS_A, ROWS_A)])

        @pl.when(sid == NS - 1)
        def _():
            pltpu.sync_copy(zeros_hbm.at[pl.ds(TAIL_LO, TAIL_N)],
                            shared.at[pl.ds(TAIL_LO, TAIL_N)])

        plsc.subcore_barrier()

        def body(k, carry):
            base = (wid * CHUNKS_PER_TILE + k) * CHUNK
            pltpu.sync_copy(dst_hbm.at[pl.ds(base, CHUNK)], idx_v)
            pltpu.sync_copy(ones_v, shared.at[idx_v], add=True)
            return carry

        lax.fori_loop(0, CHUNKS_PER_TILE, body, 0)

        @pl.when(wid < EXTRA)
        def _():
            base = (CHUNKS_PER_TILE * NW + wid) * CHUNK
            pltpu.sync_copy(dst_hbm.at[pl.ds(base, CHUNK)], idx_v)
            pltpu.sync_copy(ones_v, shared.at[idx_v], add=True)

        plsc.subcore_barrier()
        pltpu.sync_copy(shared.at[pl.ds(sid * ROWS_A, ROWS_A)],
                        out_hbm.at[cid, pl.ds(sid * ROWS_A, ROWS_A)])

        @pl.when(sid == NS - 1)
        def _():
            pltpu.sync_copy(shared.at[pl.ds(TAIL_LO, TAIL_N)],
                            out_hbm.at[cid, pl.ds(TAIL_LO, TAIL_N)])


@functools.cache
def _sc_deg():
    return pl.kernel(
        _sc_deg_body,
        out_type=jax.ShapeDtypeStruct((NC, N, 16), jnp.float32),
        mesh=_mesh(),
        scratch_types=[
            pltpu.VMEM((CHUNK,), jnp.int32),
            pltpu.VMEM((CHUNK, 16), jnp.float32),
            pltpu.SemaphoreType.DMA,
            pltpu.VMEM_SHARED((N, 16), jnp.float32),
        ],
    )


NK = CHUNKS_PER_TILE
IB = 4


NG = NK // 2


def _sc_agg_body(xs_hbm, src3_hbm, dst2_hbm, esrc_hbm, edst_hbm, zeros_hbm,
                 out_hbm, bsrc, bdst, rows_v, semi, semg, shared):
    cid = lax.axis_index("c")
    sid = lax.axis_index("s")
    wid = sid * NC + cid

    pltpu.async_copy(src3_hbm.at[wid], bsrc, semi)

    pltpu.sync_copy(zeros_hbm.at[pl.ds(sid * ROWS_A, ROWS_A)],
                    shared.at[pl.ds(sid * ROWS_A, ROWS_A)])

    @pl.when(sid == NS - 1)
    def _():
        pltpu.sync_copy(zeros_hbm.at[pl.ds(TAIL_LO, TAIL_N)],
                        shared.at[pl.ds(TAIL_LO, TAIL_N)])

    pltpu.make_async_copy(src3_hbm.at[0], bsrc, semi).wait()
    plsc.subcore_barrier()

    rbase = wid * NK

    def body(k, carry):
        pltpu.sync_copy(dst2_hbm.at[rbase + k], bdst.at[0])
        pltpu.async_copy(xs_hbm.at[bsrc.at[k]], rows_v, semg)
        pltpu.make_async_copy(xs_hbm.at[bsrc.at[0]], rows_v, semg).wait()
        pltpu.sync_copy(rows_v, shared.at[bdst.at[0]], add=True)
        return carry

    lax.fori_loop(0, NK, body, 0)

    @pl.when(wid < EXTRA)
    def _():
        pltpu.sync_copy(edst_hbm.at[wid], bdst.at[0])
        pltpu.sync_copy(esrc_hbm.at[wid], bsrc.at[0])
        pltpu.async_copy(xs_hbm.at[bsrc.at[0]], rows_v, semg)
        pltpu.make_async_copy(xs_hbm.at[bsrc.at[0]], rows_v, semg).wait()
        pltpu.sync_copy(rows_v, shared.at[bdst.at[0]], add=True)

    plsc.subcore_barrier()
    pltpu.sync_copy(shared.at[pl.ds(sid * ROWS_A, ROWS_A)],
                    out_hbm.at[cid, pl.ds(sid * ROWS_A, ROWS_A)])

    @pl.when(sid == NS - 1)
    def _():
        pltpu.sync_copy(shared.at[pl.ds(TAIL_LO, TAIL_N)],
                        out_hbm.at[cid, pl.ds(TAIL_LO, TAIL_N)])


@functools.cache
def _sc_agg():
    return pl.kernel(
        _sc_agg_body,
        out_type=jax.ShapeDtypeStruct((NC, N, D), jnp.float32),
        mesh=_mesh(),
        scratch_types=[
            pltpu.VMEM((NK, CHUNK), jnp.int32),
            pltpu.VMEM((4, CHUNK), jnp.int32),
            pltpu.VMEM((CHUNK, D), jnp.float32),
            pltpu.SemaphoreType.DMA,
            pltpu.SemaphoreType.DMA,
            pltpu.VMEM_SHARED((N, D), jnp.float32),
        ], )


_RB = 1000
_GRID = N // _RB


def _dis(deg_ref):
    return lax.rsqrt(deg_ref[0, :, 0:1] + deg_ref[1, :, 0:1] + 1.0)


def _xw_body(x_ref, w_ref, deg_ref, o_ref):
    o_ref[...] = jnp.dot(x_ref[...], w_ref[...],
                         preferred_element_type=jnp.float32) * _dis(deg_ref)


def _fuse_body(p_ref, xs_ref, deg_ref, b_ref, g_ref, be_ref, w_ref, o_ref):
    dis = _dis(deg_ref)
    agg = (p_ref[0] + p_ref[1] + xs_ref[...]) * dis + b_ref[...]
    h = jnp.maximum(agg, 0.0) * (g_ref[...] * BN_INV) + be_ref[...]
    o_ref[...] = jnp.dot(h, w_ref[...],
                         preferred_element_type=jnp.float32) * dis


def _final_body(p_ref, xs_ref, deg_ref, b_ref, act_ref, fw1_ref, fb1_ref,
                g3_ref, be3_ref, fw2_ref, fb2_ref, fw3_ref, fb3_ref,
                o_ref, acc):
    i = pl.program_id(0)
    dis = _dis(deg_ref)
    h = jnp.maximum((p_ref[0] + p_ref[1] + xs_ref[...]) * dis + b_ref[...], 0.0)
    s = jnp.sum(h, axis=0, keepdims=True)

    @pl.when(i == 0)
    def _():
        acc[...] = s

    @pl.when(i > 0)
    def _():
        acc[...] = acc[...] + s

    @pl.when(i == _GRID - 1)
    def _():
        pooled = jnp.concatenate(
            [acc[...] * (1.0 / N), jnp.zeros((7, 128), jnp.float32)], axis=0)
        z = (jnp.dot(pooled, fw1_ref[0:128, :], preferred_element_type=jnp.float32)
             + jnp.dot(act_ref[...], fw1_ref[128:144, :],
                       preferred_element_type=jnp.float32)
             + fb1_ref[...])
        z = jnp.maximum(z, 0.0) * (g3_ref[...] * BN_INV) + be3_ref[...]
        z = jnp.maximum(jnp.dot(z, fw2_ref[...],
                                preferred_element_type=jnp.float32)
                        + fb2_ref[...], 0.0)
        z = jnp.dot(z, fw3_ref[...], preferred_element_type=jnp.float32) + fb3_ref[...]
        o_ref[...] = z[0:1, :]


def _full(shape):
    nd = len(shape)
    return pl.BlockSpec(shape, lambda i, _n=nd: (0,) * _n)


_row_spec = pl.BlockSpec((_RB, D), lambda i: (i, 0))
_p_spec = pl.BlockSpec((NC, _RB, D), lambda i: (0, i, 0))
_deg_spec = pl.BlockSpec((NC, _RB, 16), lambda i: (0, i, 0))

_tc_xw = pl.pallas_call(
    _xw_body,
    grid=(_GRID,),
    in_specs=[_row_spec, _full((D, D)), _deg_spec],
    out_specs=_row_spec,
    out_shape=jax.ShapeDtypeStruct((N, D), jnp.float32),
)

_tc_fuse = pl.pallas_call(
    _fuse_body,
    grid=(_GRID,),
    in_specs=[_p_spec, _row_spec, _deg_spec, _full((1, D)), _full((1, D)),
              _full((1, D)), _full((D, D))],
    out_specs=_row_spec,
    out_shape=jax.ShapeDtypeStruct((N, D), jnp.float32),
)

_tc_final = pl.pallas_call(
    _final_body,
    grid=(_GRID,),
    in_specs=[_p_spec, _row_spec, _deg_spec, _full((1, D)), _full((8, 16)),
              _full((303, D)), _full((1, D)), _full((1, D)), _full((1, D)),
              _full((D, 64)), _full((1, 64)), _full((64, 10)), _full((1, 10))],
    out_specs=_full((1, 10)),
    out_shape=jax.ShapeDtypeStruct((1, 10), jnp.float32),
    scratch_shapes=[pltpu.VMEM((1, D), jnp.float32)],
)


def kernel(x, edge_index, actions, W1, b1, W2, b2, W3, b3, g1, be1, g2, be2,
           g3, be3, fcW1, fcb1, fcW2, fcb2, fcW3, fcb3):
    src = edge_index[0]
    dst = edge_index[1]
    eb = NW * NK * CHUNK
    src3 = src[:eb].reshape(NW, NK, CHUNK)
    dst2 = dst[:eb].reshape(NW * NK, CHUNK)
    esrc = src[eb:].reshape(EXTRA, CHUNK)
    edst = dst[eb:].reshape(EXTRA, CHUNK)
    zeros_deg = jnp.zeros((N, 16), jnp.float32)
    ones_chunk = jnp.ones((CHUNK, 16), jnp.float32)
    zeros_big = jnp.zeros((N, D), jnp.float32)
    act8 = jnp.pad(actions.astype(jnp.float32), ((0, 7), (0, 0)))
    row = lambda v: v.reshape(1, -1)

    degp = _sc_deg()(dst, zeros_deg, ones_chunk)

    xs1 = _tc_xw(x, W1, degp)
    p1 = _sc_agg()(xs1, src3, dst2, esrc, edst, zeros_big)
    xs2 = _tc_fuse(p1, xs1, degp, row(b1), row(g1), row(be1), W2)
    p2 = _sc_agg()(xs2, src3, dst2, esrc, edst, zeros_big)
    xs3 = _tc_fuse(p2, xs2, degp, row(b2), row(g2), row(be2), W3)
    p3 = _sc_agg()(xs3, src3, dst2, esrc, edst, zeros_big)

    return _tc_final(p3, xs3, degp, row(b3), act8, fcW1, row(fcb1),
                     row(g3), row(be3), fcW2, row(fcb2), fcW3, row(fcb3))
